# baseline (speedup 1.0000x reference)
import functools

import jax
import jax.numpy as jnp
from jax import lax
from jax.experimental import pallas as pl
from jax.experimental.pallas import tpu as pltpu
from jax.experimental.pallas import tpu_sc as plsc

Q = 65536
D = 128
B = 1024
NW = 32
RPW = Q // NW
CH = 512
NCH = RPW // CH
SH = Q - B

_mesh = plsc.VectorSubcoreMesh(core_axis_name="c", subcore_axis_name="s")


@functools.partial(
    pl.kernel,
    mesh=_mesh,
    out_type=jax.ShapeDtypeStruct((Q, D), jnp.float32),
    scratch_types=[
        pltpu.VMEM((CH, D), jnp.float32),
        pltpu.VMEM((CH, D), jnp.float32),
        pltpu.SemaphoreType.DMA,
        pltpu.SemaphoreType.DMA,
        pltpu.SemaphoreType.DMA,
        pltpu.SemaphoreType.DMA,
    ],
)
def _sc_copy(keys_hbm, old_hbm, b0, b1, sr0, sr1, sw0, sw1):
    wid = lax.axis_index("s") * 2 + lax.axis_index("c")
    base = wid * RPW
    bufs = (b0, b1)
    srs = (sr0, sr1)
    sws = (sw0, sw1)

    reads = {0: pltpu.async_copy(keys_hbm.at[pl.ds(base, CH)], bufs[0], srs[0])}
    writes = {}
    for c in range(NCH):
        bsel = c % 2
        reads[c].wait()
        writes[c] = pltpu.async_copy(
            bufs[bsel], old_hbm.at[pl.ds(base + c * CH, CH)], sws[bsel])
        if c + 1 < NCH:
            nb = (c + 1) % 2
            if c >= 1:
                writes[c - 1].wait()
            reads[c + 1] = pltpu.async_copy(
                keys_hbm.at[pl.ds(base + (c + 1) * CH, CH)], bufs[nb], srs[nb])
    writes[NCH - 2].wait()
    writes[NCH - 1].wait()


def _tc_grid_body(new_ref, keys_ref, out_ref):
    i = pl.program_id(0)
    out_ref[...] = jnp.where(i == 0, new_ref[...], keys_ref[...])


_tc_shift = pl.pallas_call(
    _tc_grid_body,
    grid=(Q // B,),
    in_specs=[
        pl.BlockSpec((B, D), lambda i: (0, 0)),
        pl.BlockSpec((B, D), lambda i: (jnp.maximum(i - 1, 0), 0)),
    ],
    out_specs=pl.BlockSpec((B, D), lambda i: (i, 0)),
    out_shape=jax.ShapeDtypeStruct((Q, D), jnp.float32),
)


def kernel(new_keys, keys):
    old_keys = _sc_copy(keys)
    updated_keys = _tc_shift(new_keys, keys)
    return (old_keys, updated_keys)


# speedup vs baseline: 1.4402x; 1.4402x over previous
import functools

import jax
import jax.numpy as jnp
from jax import lax
from jax.experimental import pallas as pl
from jax.experimental.pallas import tpu as pltpu
from jax.experimental.pallas import tpu_sc as plsc

Q = 65536
D = 128
B = 1024
NW = 32
SH = Q - B
SPW = SH // NW
CH = 336
NCH = SPW // CH
SGL = (2 * B) // NW

_mesh = plsc.VectorSubcoreMesh(core_axis_name="c", subcore_axis_name="s")


@functools.partial(
    pl.kernel,
    mesh=_mesh,
    out_type=(
        jax.ShapeDtypeStruct((Q, D), jnp.float32),
        jax.ShapeDtypeStruct((Q, D), jnp.float32),
    ),
    scratch_types=[
        pltpu.VMEM((CH, D), jnp.float32),
        pltpu.VMEM((CH, D), jnp.float32),
        pltpu.VMEM((SGL, D), jnp.float32),
        pltpu.SemaphoreType.DMA,
        pltpu.SemaphoreType.DMA,
        pltpu.SemaphoreType.DMA,
        pltpu.SemaphoreType.DMA,
        pltpu.SemaphoreType.DMA,
    ],
)
def _fifo_shift(new_hbm, keys_hbm, old_hbm, upd_hbm,
                b0, b1, sb, sr0, sr1, sw0, sw1, ss):
    wid = lax.axis_index("s") * 2 + lax.axis_index("c")
    base = wid * SPW
    bufs = (b0, b1)
    srs = (sr0, sr1)
    sws = (sw0, sw1)
    half = NW // 2

    @pl.when(wid < half)
    def _():
        pltpu.async_copy(keys_hbm.at[pl.ds(SH + wid * SGL, SGL)], sb, ss)

    @pl.when(wid >= half)
    def _():
        pltpu.async_copy(new_hbm.at[pl.ds((wid - half) * SGL, SGL)], sb, ss)

    reads = {0: pltpu.async_copy(keys_hbm.at[pl.ds(base, CH)], bufs[0], srs[0])}
    writes = {}
    for c in range(NCH):
        bsel = c % 2
        reads[c].wait()
        off = base + c * CH
        writes[c] = (
            pltpu.async_copy(bufs[bsel], old_hbm.at[pl.ds(off, CH)], sws[bsel]),
            pltpu.async_copy(bufs[bsel], upd_hbm.at[pl.ds(off + B, CH)], sws[bsel]),
        )
        if c + 1 < NCH:
            nb = (c + 1) % 2
            if c >= 1:
                writes[c - 1][0].wait()
                writes[c - 1][1].wait()
            reads[c + 1] = pltpu.async_copy(
                keys_hbm.at[pl.ds(base + (c + 1) * CH, CH)], bufs[nb], srs[nb])

    pltpu.make_async_copy(keys_hbm.at[pl.ds(0, SGL)], sb, ss).wait()

    @pl.when(wid < half)
    def _():
        pltpu.async_copy(sb, old_hbm.at[pl.ds(SH + wid * SGL, SGL)], ss)

    @pl.when(wid >= half)
    def _():
        pltpu.async_copy(sb, upd_hbm.at[pl.ds((wid - half) * SGL, SGL)], ss)

    writes[NCH - 2][0].wait()
    writes[NCH - 2][1].wait()
    writes[NCH - 1][0].wait()
    writes[NCH - 1][1].wait()
    pltpu.make_async_copy(keys_hbm.at[pl.ds(0, SGL)], sb, ss).wait()


def kernel(new_keys, keys):
    old_keys, updated_keys = _fifo_shift(new_keys, keys)
    return (old_keys, updated_keys)
